# Initial kernel scaffold; baseline (speedup 1.0000x reference)
import functools

import jax
import jax.numpy as jnp
from jax import lax
from jax.experimental import pallas as pl
from jax.experimental.pallas import tpu as pltpu
from jax.experimental.pallas import tpu_sc as plsc

N = 10000
E = 320000
D_IN = 128
D_H = 128
D_OUT = 16
DA = 144

C = 128
NCHUNK = E // C
NW = 32
ROWS_PER_TILE = N // 16
ZCHUNK = 125

BM = 1000


def _make_sc_segsum(width):
    mesh = plsc.VectorSubcoreMesh(core_axis_name="c", subcore_axis_name="s")

    @functools.partial(
        pl.kernel,
        mesh=mesh,
        out_type=jax.ShapeDtypeStruct((2 * N, width), jnp.float32),
        scratch_types=[
            pltpu.VMEM((C,), jnp.int32),
            pltpu.VMEM((C,), jnp.int32),
            pltpu.VMEM((C, width), jnp.float32),
            pltpu.VMEM_SHARED((N, width), jnp.float32),
            pltpu.SemaphoreType.DMA,
        ],
    )
    def segsum(src_hbm, dst_hbm, proj_hbm, zeros_hbm, out_hbm,
               sidx, didx, rows, acc, sem):
        cid = lax.axis_index("c")
        sid = lax.axis_index("s")
        wid = cid * 16 + sid

        pltpu.sync_copy(zeros_hbm, rows.at[pl.ds(0, ZCHUNK)])
        row0 = sid * ROWS_PER_TILE
        for k in range(5):
            pltpu.sync_copy(rows.at[pl.ds(0, ZCHUNK)],
                            acc.at[pl.ds(row0 + k * ZCHUNK, ZCHUNK)])
        plsc.subcore_barrier()

        nj = (NCHUNK - wid + NW - 1) // NW

        def body(t, carry):
            base = (wid + t * NW) * C
            pltpu.sync_copy(src_hbm.at[pl.ds(base, C)], sidx)
            pltpu.sync_copy(dst_hbm.at[pl.ds(base, C)], didx)
            pltpu.async_copy(proj_hbm.at[sidx], rows, sem).wait()
            pltpu.sync_copy(rows, acc.at[didx], add=True)
            return carry

        lax.fori_loop(0, nj, body, 0)
        plsc.subcore_barrier()

        for k in range(5):
            r = row0 + k * ZCHUNK
            pltpu.sync_copy(acc.at[pl.ds(r, ZCHUNK)], rows.at[pl.ds(0, ZCHUNK)])
            pltpu.sync_copy(rows.at[pl.ds(0, ZCHUNK)],
                            out_hbm.at[pl.ds(cid * N + r, ZCHUNK)])

    return segsum


_sc_segsum_l1 = _make_sc_segsum(DA)
_sc_segsum_l2 = _make_sc_segsum(D_OUT)


def _tc1_body(f_ref, w_ref, o_ref):
    p = jnp.dot(f_ref[...], w_ref[...], preferred_element_type=jnp.float32)
    col = lax.broadcasted_iota(jnp.int32, (BM, DA - D_H), 1)
    aug = jnp.where(col == 0, 1.0, 0.0)
    o_ref[...] = jnp.concatenate([p, aug], axis=1)


def _tc2_body(f_ref, pa_ref, pb_ref, w1s_ref, b1_ref, w2n_ref, w2s_ref, b2_ref,
              p2_ref, hs2_ref, dinv_ref):
    s = pa_ref[...] + pb_ref[...]
    deg = jnp.clip(s[:, D_H:D_H + 1], 1.0, None)
    dinv = 1.0 / deg
    agg1 = s[:, :D_H] * dinv
    h = jnp.dot(f_ref[...], w1s_ref[...], preferred_element_type=jnp.float32)
    h = jnp.maximum(h + agg1 + b1_ref[0:1, :], 0.0)
    p2_ref[...] = jnp.dot(h, w2n_ref[...], preferred_element_type=jnp.float32)
    hs2_ref[...] = (
        jnp.dot(h, w2s_ref[...], preferred_element_type=jnp.float32)
        + b2_ref[0:1, :]
    )
    dinv_ref[...] = jnp.broadcast_to(dinv, (BM, D_OUT))


def _tc3_body(hs2_ref, qa_ref, qb_ref, dinv_ref, o_ref):
    o_ref[...] = hs2_ref[...] + (qa_ref[...] + qb_ref[...]) * dinv_ref[...]


def kernel(features, edge_index, W1_self, W1_neigh, b1, W2_self, W2_neigh, b2):
    src = edge_index[0]
    dst = edge_index[1]
    nb = N // BM

    p1aug = pl.pallas_call(
        _tc1_body,
        grid=(nb,),
        in_specs=[
            pl.BlockSpec((BM, D_IN), lambda i: (i, 0)),
            pl.BlockSpec((D_IN, D_H), lambda i: (0, 0)),
        ],
        out_specs=pl.BlockSpec((BM, DA), lambda i: (i, 0)),
        out_shape=jax.ShapeDtypeStruct((N, DA), jnp.float32),
    )(features, W1_neigh)

    zeros1 = jnp.zeros((ZCHUNK, DA), jnp.float32)
    part1 = _sc_segsum_l1(src, dst, p1aug, zeros1)

    b1_2d = jnp.broadcast_to(b1[None, :], (8, D_H))
    b2_2d = jnp.broadcast_to(b2[None, :], (8, D_OUT))
    p2, hs2, dinv = pl.pallas_call(
        _tc2_body,
        grid=(nb,),
        in_specs=[
            pl.BlockSpec((BM, D_IN), lambda i: (i, 0)),
            pl.BlockSpec((BM, DA), lambda i: (i, 0)),
            pl.BlockSpec((BM, DA), lambda i: (i + nb, 0)),
            pl.BlockSpec((D_IN, D_H), lambda i: (0, 0)),
            pl.BlockSpec((8, D_H), lambda i: (0, 0)),
            pl.BlockSpec((D_H, D_OUT), lambda i: (0, 0)),
            pl.BlockSpec((D_H, D_OUT), lambda i: (0, 0)),
            pl.BlockSpec((8, D_OUT), lambda i: (0, 0)),
        ],
        out_specs=[
            pl.BlockSpec((BM, D_OUT), lambda i: (i, 0)),
            pl.BlockSpec((BM, D_OUT), lambda i: (i, 0)),
            pl.BlockSpec((BM, D_OUT), lambda i: (i, 0)),
        ],
        out_shape=[
            jax.ShapeDtypeStruct((N, D_OUT), jnp.float32),
            jax.ShapeDtypeStruct((N, D_OUT), jnp.float32),
            jax.ShapeDtypeStruct((N, D_OUT), jnp.float32),
        ],
    )(features, part1, part1, W1_self, b1_2d, W2_neigh, W2_self, b2_2d)

    zeros2 = jnp.zeros((ZCHUNK, D_OUT), jnp.float32)
    part2 = _sc_segsum_l2(src, dst, p2, zeros2)

    out = pl.pallas_call(
        _tc3_body,
        grid=(nb,),
        in_specs=[
            pl.BlockSpec((BM, D_OUT), lambda i: (i, 0)),
            pl.BlockSpec((BM, D_OUT), lambda i: (i, 0)),
            pl.BlockSpec((BM, D_OUT), lambda i: (i + nb, 0)),
            pl.BlockSpec((BM, D_OUT), lambda i: (i, 0)),
        ],
        out_specs=pl.BlockSpec((BM, D_OUT), lambda i: (i, 0)),
        out_shape=jax.ShapeDtypeStruct((N, D_OUT), jnp.float32),
    )(hs2, part2, part2, dinv)

    return out


# speedup vs baseline: 6.5533x; 6.5533x over previous
import functools

import jax
import jax.numpy as jnp
from jax import lax
from jax.experimental import pallas as pl
from jax.experimental.pallas import tpu as pltpu
from jax.experimental.pallas import tpu_sc as plsc

N = 10000
E = 320000
D_IN = 128
D_H = 128
D_OUT = 16

C = 128
NCHUNK = E // C
NW = 32
NZFULL = N // C
NZREM = N - NZFULL * C

BM = 1000


def _make_sc_segsum(width, gather):
    mesh = plsc.VectorSubcoreMesh(core_axis_name="c", subcore_axis_name="s")
    nconst = 256 if not gather else 128

    def segsum(*args):
        if gather:
            (src_hbm, dst_hbm, proj_hbm, const_hbm, out_hbm,
             sidx, didx, rows, acc, sem) = args
        else:
            (dst_hbm, const_hbm, out_hbm, sidx, didx, rows, acc, sem) = args
        cid = lax.axis_index("c")
        sid = lax.axis_index("s")
        wid = cid * 16 + sid

        pltpu.sync_copy(const_hbm.at[pl.ds(0, C)], rows)
        for t in range(5):
            j = sid + 16 * t

            @pl.when(j < NZFULL)
            def _():
                pltpu.sync_copy(rows, acc.at[pl.ds(pl.multiple_of(j * C, C), C)])

        @pl.when(sid == 15)
        def _():
            pltpu.sync_copy(rows.at[pl.ds(0, NZREM)],
                            acc.at[pl.ds(NZFULL * C, NZREM)])

        if not gather:
            pltpu.sync_copy(const_hbm.at[pl.ds(C, C)], rows)
        plsc.subcore_barrier()

        nj = (NCHUNK - wid + NW - 1) // NW

        def body(t, carry):
            base = pl.multiple_of((wid + t * NW) * C, C)
            pltpu.sync_copy(dst_hbm.at[pl.ds(base, C)], didx)
            if gather:
                pltpu.sync_copy(src_hbm.at[pl.ds(base, C)], sidx)
                pltpu.async_copy(proj_hbm.at[sidx], rows, sem).wait()
            pltpu.sync_copy(rows, acc.at[didx], add=True)
            return carry

        lax.fori_loop(0, nj, body, 0)
        plsc.subcore_barrier()

        for t in range(5):
            j = sid + 16 * t

            @pl.when(j < NZFULL)
            def _():
                r = pl.multiple_of(j * C, C)
                pltpu.sync_copy(acc.at[pl.ds(r, C)], rows)
                pltpu.sync_copy(
                    rows, out_hbm.at[pl.ds(pl.multiple_of(cid * N + r, 16), C)])

        @pl.when(sid == 15)
        def _():
            pltpu.sync_copy(acc.at[pl.ds(NZFULL * C, NZREM)],
                            rows.at[pl.ds(0, NZREM)])
            pltpu.sync_copy(
                rows.at[pl.ds(0, NZREM)],
                out_hbm.at[pl.ds(pl.multiple_of(cid * N + NZFULL * C, 16),
                                 NZREM)])

    return functools.partial(
        pl.kernel,
        mesh=mesh,
        compiler_params=pltpu.CompilerParams(use_tc_tiling_on_sc=False),
        out_type=jax.ShapeDtypeStruct((2 * N, width), jnp.float32),
        scratch_types=[
            pltpu.VMEM((C,), jnp.int32),
            pltpu.VMEM((C,), jnp.int32),
            pltpu.VMEM((C, width), jnp.float32),
            pltpu.VMEM_SHARED((N, width), jnp.float32),
            pltpu.SemaphoreType.DMA,
        ],
    )(segsum)


_sc_segsum_l1 = _make_sc_segsum(D_H, gather=True)
_sc_segsum_l2 = _make_sc_segsum(D_OUT, gather=True)
_sc_degree = _make_sc_segsum(D_OUT, gather=False)


def _tc1_body(f_ref, w_ref, o_ref):
    o_ref[...] = jnp.dot(f_ref[...], w_ref[...],
                         preferred_element_type=jnp.float32)


def _tc2_body(f_ref, pa_ref, pb_ref, da_ref, db_ref,
              w1s_ref, b1_ref, w2n_ref, w2s_ref, b2_ref,
              p2_ref, hs2_ref, dinv_ref):
    deg = da_ref[:, 0:1] + db_ref[:, 0:1]
    dinv = 1.0 / jnp.clip(deg, 1.0, None)
    agg1 = (pa_ref[...] + pb_ref[...]) * dinv
    h = jnp.dot(f_ref[...], w1s_ref[...], preferred_element_type=jnp.float32)
    h = jnp.maximum(h + agg1 + b1_ref[0:1, :], 0.0)
    p2_ref[...] = jnp.dot(h, w2n_ref[...], preferred_element_type=jnp.float32)
    hs2_ref[...] = (
        jnp.dot(h, w2s_ref[...], preferred_element_type=jnp.float32)
        + b2_ref[0:1, :]
    )
    dinv_ref[...] = jnp.broadcast_to(dinv, (BM, D_OUT))


def _tc3_body(hs2_ref, qa_ref, qb_ref, dinv_ref, o_ref):
    o_ref[...] = hs2_ref[...] + (qa_ref[...] + qb_ref[...]) * dinv_ref[...]


def kernel(features, edge_index, W1_self, W1_neigh, b1, W2_self, W2_neigh, b2):
    src = edge_index[0]
    dst = edge_index[1]
    nb = N // BM

    p1 = pl.pallas_call(
        _tc1_body,
        grid=(nb,),
        in_specs=[
            pl.BlockSpec((BM, D_IN), lambda i: (i, 0)),
            pl.BlockSpec((D_IN, D_H), lambda i: (0, 0)),
        ],
        out_specs=pl.BlockSpec((BM, D_H), lambda i: (i, 0)),
        out_shape=jax.ShapeDtypeStruct((N, D_H), jnp.float32),
    )(features, W1_neigh)

    const_deg = jnp.concatenate(
        [jnp.zeros((C, D_OUT), jnp.float32), jnp.ones((C, D_OUT), jnp.float32)])
    part_deg = _sc_degree(dst, const_deg)

    zeros1 = jnp.zeros((C, D_H), jnp.float32)
    part1 = _sc_segsum_l1(src, dst, p1, zeros1)

    b1_2d = jnp.broadcast_to(b1[None, :], (8, D_H))
    b2_2d = jnp.broadcast_to(b2[None, :], (8, D_OUT))
    p2, hs2, dinv = pl.pallas_call(
        _tc2_body,
        grid=(nb,),
        in_specs=[
            pl.BlockSpec((BM, D_IN), lambda i: (i, 0)),
            pl.BlockSpec((BM, D_H), lambda i: (i, 0)),
            pl.BlockSpec((BM, D_H), lambda i: (i + nb, 0)),
            pl.BlockSpec((BM, D_OUT), lambda i: (i, 0)),
            pl.BlockSpec((BM, D_OUT), lambda i: (i + nb, 0)),
            pl.BlockSpec((D_IN, D_H), lambda i: (0, 0)),
            pl.BlockSpec((8, D_H), lambda i: (0, 0)),
            pl.BlockSpec((D_H, D_OUT), lambda i: (0, 0)),
            pl.BlockSpec((D_H, D_OUT), lambda i: (0, 0)),
            pl.BlockSpec((8, D_OUT), lambda i: (0, 0)),
        ],
        out_specs=[
            pl.BlockSpec((BM, D_OUT), lambda i: (i, 0)),
            pl.BlockSpec((BM, D_OUT), lambda i: (i, 0)),
            pl.BlockSpec((BM, D_OUT), lambda i: (i, 0)),
        ],
        out_shape=[
            jax.ShapeDtypeStruct((N, D_OUT), jnp.float32),
            jax.ShapeDtypeStruct((N, D_OUT), jnp.float32),
            jax.ShapeDtypeStruct((N, D_OUT), jnp.float32),
        ],
    )(features, part1, part1, part_deg, part_deg,
      W1_self, b1_2d, W2_neigh, W2_self, b2_2d)

    zeros2 = jnp.zeros((C, D_OUT), jnp.float32)
    part2 = _sc_segsum_l2(src, dst, p2, zeros2)

    out = pl.pallas_call(
        _tc3_body,
        grid=(nb,),
        in_specs=[
            pl.BlockSpec((BM, D_OUT), lambda i: (i, 0)),
            pl.BlockSpec((BM, D_OUT), lambda i: (i, 0)),
            pl.BlockSpec((BM, D_OUT), lambda i: (i + nb, 0)),
            pl.BlockSpec((BM, D_OUT), lambda i: (i, 0)),
        ],
        out_specs=pl.BlockSpec((BM, D_OUT), lambda i: (i, 0)),
        out_shape=jax.ShapeDtypeStruct((N, D_OUT), jnp.float32),
    )(hs2, part2, part2, dinv)

    return out
